# baseline (speedup 1.0000x reference)
import functools

import jax
import jax.numpy as jnp
from jax import lax
from jax.experimental import pallas as pl
from jax.experimental.pallas import tpu as pltpu
from jax.experimental.pallas import tpu_sc as plsc

_N = 10000
_E = 320000
_DN = 128
_DE = 128
_NDICT = 20
_EDICT = 400
_NC = 150
_CLOW = 0.0
_CHIGH = 30.0
_NCONV = 3

_EB = 2560
_NB = 1000
_NWORK = 32
_EPW = _E // _NWORK
_CHK = 80
_NCHUNK = _EPW // _CHK
_NPAD = 10112
_RPT = _NPAD // 16


def _estat_body(dist_ref, etype_ref, etab_ref, we_ref, wr_ref, bmsg_ref,
                out_ref, te_ref):
    i = pl.program_id(0)

    @pl.when(i == 0)
    def _():
        te_ref[...] = jnp.dot(etab_ref[...], we_ref[...],
                              preferred_element_type=jnp.float32
                              ).astype(jnp.bfloat16)

    delta = (_CHIGH - _CLOW) / (_NC - 1)
    centers = _CLOW + delta * lax.broadcasted_iota(
        jnp.int32, (1, _NC), 1).astype(jnp.float32)
    d = dist_ref[...]
    z = (d - centers) * (1.0 / delta)
    rbf = jnp.exp(-(z * z)).astype(jnp.bfloat16)
    r_part = jnp.dot(rbf, wr_ref[...].astype(jnp.bfloat16),
                     preferred_element_type=jnp.float32)
    t = etype_ref[...]
    oh = (t == lax.broadcasted_iota(jnp.int32, (1, _EDICT), 1))
    e_part = jnp.dot(oh.astype(jnp.bfloat16), te_ref[...],
                     preferred_element_type=jnp.float32)
    out_ref[...] = r_part + e_part + bmsg_ref[...]


def _estat_call(dist2, etype2, edge_table, w_e, w_r, bmsg2):
    return pl.pallas_call(
        _estat_body,
        grid=(_E // _EB,),
        in_specs=[
            pl.BlockSpec((_EB, 1), lambda i: (i, 0)),
            pl.BlockSpec((_EB, 1), lambda i: (i, 0)),
            pl.BlockSpec((_EDICT, _DN), lambda i: (0, 0)),
            pl.BlockSpec((_DE, _DN), lambda i: (0, 0)),
            pl.BlockSpec((_NC, _DN), lambda i: (0, 0)),
            pl.BlockSpec((1, _DN), lambda i: (0, 0)),
        ],
        out_specs=pl.BlockSpec((_EB, _DN), lambda i: (i, 0)),
        out_shape=jax.ShapeDtypeStruct((_E, _DN), jnp.float32),
        scratch_shapes=[pltpu.VMEM((_EDICT, _DN), jnp.bfloat16)],
    )(dist2, etype2, edge_table, w_e, w_r, bmsg2)


def _hw0_body(z_ref, ntab_ref, wh_ref, out_ref):
    twh = jnp.dot(ntab_ref[...], wh_ref[...],
                  preferred_element_type=jnp.float32)
    oh = (z_ref[...] == lax.broadcasted_iota(jnp.int32, (1, _NDICT), 1))
    out_ref[...] = jnp.dot(oh.astype(jnp.float32), twh,
                           preferred_element_type=jnp.float32)


def _hw0_call(z2, node_table, w_h):
    return pl.pallas_call(
        _hw0_body,
        grid=(_N // _NB,),
        in_specs=[
            pl.BlockSpec((_NB, 1), lambda i: (i, 0)),
            pl.BlockSpec((_NDICT, _DN), lambda i: (0, 0)),
            pl.BlockSpec((_DN, _DN), lambda i: (0, 0)),
        ],
        out_specs=pl.BlockSpec((_NB, _DN), lambda i: (i, 0)),
        out_shape=jax.ShapeDtypeStruct((_N, _DN), jnp.float32),
    )(z2, node_table, w_h)


def _upd_body(p0_ref, p1_ref, wu_ref, bu_ref, wh_ref, h_ref, hw_ref):
    agg = p0_ref[...] + p1_ref[...]
    h = jnp.maximum(
        jnp.dot(agg, wu_ref[...], preferred_element_type=jnp.float32)
        + bu_ref[...], 0.0)
    h_ref[...] = h
    hw_ref[...] = jnp.dot(h, wh_ref[...], preferred_element_type=jnp.float32)


def _upd_call(p0, p1, w_upd, bu2, w_h):
    return pl.pallas_call(
        _upd_body,
        grid=(_N // _NB,),
        in_specs=[
            pl.BlockSpec((_NB, _DN), lambda i: (i, 0)),
            pl.BlockSpec((_NB, _DN), lambda i: (i, 0)),
            pl.BlockSpec((_DN, _DN), lambda i: (0, 0)),
            pl.BlockSpec((1, _DN), lambda i: (0, 0)),
            pl.BlockSpec((_DN, _DN), lambda i: (0, 0)),
        ],
        out_specs=[
            pl.BlockSpec((_NB, _DN), lambda i: (i, 0)),
            pl.BlockSpec((_NB, _DN), lambda i: (i, 0)),
        ],
        out_shape=[
            jax.ShapeDtypeStruct((_N, _DN), jnp.float32),
            jax.ShapeDtypeStruct((_N, _DN), jnp.float32),
        ],
    )(p0, p1, w_upd, bu2, w_h)


def _read_body(h_ref, w1_ref, b1_ref, w2_ref, b2_ref, out_ref):
    i = pl.program_id(0)
    t = jnp.maximum(
        jnp.dot(h_ref[...], w1_ref[...], preferred_element_type=jnp.float32)
        + b1_ref[...], 0.0)
    r = jnp.dot(t, w2_ref[...], preferred_element_type=jnp.float32) + b2_ref[...]
    s = jnp.sum(r)

    @pl.when(i == 0)
    def _():
        out_ref[...] = jnp.zeros((1, 1), jnp.float32)

    out_ref[...] = out_ref[...] + jnp.reshape(s, (1, 1))


def _read_call(h, w_fc1, b1_2, w_fc2, b2_2):
    return pl.pallas_call(
        _read_body,
        grid=(_N // _NB,),
        in_specs=[
            pl.BlockSpec((_NB, _DN), lambda i: (i, 0)),
            pl.BlockSpec((_DN, _DN), lambda i: (0, 0)),
            pl.BlockSpec((1, _DN), lambda i: (0, 0)),
            pl.BlockSpec((_DN, 1), lambda i: (0, 0)),
            pl.BlockSpec((1, 1), lambda i: (0, 0)),
        ],
        out_specs=pl.BlockSpec((1, 1), lambda i: (0, 0)),
        out_shape=jax.ShapeDtypeStruct((1, 1), jnp.float32),
    )(h, w_fc1, b1_2, w_fc2, b2_2)


_sc_mesh = plsc.VectorSubcoreMesh(core_axis_name="c", subcore_axis_name="s")


@functools.partial(
    pl.kernel,
    mesh=_sc_mesh,
    out_type=jax.ShapeDtypeStruct((2 * _NPAD, _DN), jnp.float32),
    scratch_types=[
        pltpu.VMEM((_CHK,), jnp.int32),
        pltpu.VMEM((_CHK,), jnp.int32),
        pltpu.VMEM((_CHK,), jnp.int32),
        pltpu.VMEM((_CHK,), jnp.int32),
        pltpu.VMEM((_CHK, _DN), jnp.float32),
        pltpu.VMEM((_CHK, _DN), jnp.float32),
        pltpu.VMEM((_CHK, _DN), jnp.float32),
        pltpu.VMEM((_CHK, _DN), jnp.float32),
        pltpu.VMEM_SHARED((_NPAD, _DN), jnp.float32),
        pltpu.SemaphoreType.DMA,
        pltpu.SemaphoreType.DMA,
        pltpu.SemaphoreType.DMA,
        pltpu.SemaphoreType.DMA,
        pltpu.SemaphoreType.DMA,
        pltpu.SemaphoreType.DMA,
    ],
)
def _sc_edge(hw_hbm, estat_hbm, src1_hbm, dst1_hbm, out_hbm,
             src0_v, src1_v, dst0_v, dst1_v, rows0, rows1, est0, est1,
             acc_sh, semi0, semi1, semg0, semg1, seme0, seme1):
    cid = lax.axis_index("c")
    sid = lax.axis_index("s")
    wid = cid * 16 + sid
    ebase = wid * _EPW

    def _zrow(r, carry):
        for k in range(_DN // 16):
            est0[r, pl.ds(k * 16, 16)] = jnp.zeros((16,), jnp.float32)
        return carry

    lax.fori_loop(0, _CHK, _zrow, 0)
    for rep in range(_RPT // _CHK):
        pltpu.sync_copy(est0,
                        acc_sh.at[pl.ds(sid * _RPT + rep * _CHK, _CHK)])
    _ZTAIL = _RPT - (_RPT // _CHK) * _CHK
    if _ZTAIL:
        pltpu.sync_copy(
            est0.at[pl.ds(0, _ZTAIL)],
            acc_sh.at[pl.ds(sid * _RPT + (_RPT // _CHK) * _CHK, _ZTAIL)])
    plsc.subcore_barrier()

    rows = (rows0, rows1)
    est = (est0, est1)
    srcb = (src0_v, src1_v)
    dstb = (dst0_v, dst1_v)
    semg = (semg0, semg1)
    seme = (seme0, seme1)
    semi = (semi0, semi1)

    def _issue_idx(c, b):
        pltpu.async_copy(src1_hbm.at[pl.ds(ebase + c * _CHK, _CHK)],
                         srcb[b], semi[b])
        pltpu.async_copy(dst1_hbm.at[pl.ds(ebase + c * _CHK, _CHK)],
                         dstb[b], semi[b])

    def _wait_idx(c, b):
        pltpu.make_async_copy(src1_hbm.at[pl.ds(ebase + c * _CHK, _CHK)],
                              srcb[b], semi[b]).wait()
        pltpu.make_async_copy(dst1_hbm.at[pl.ds(ebase + c * _CHK, _CHK)],
                              dstb[b], semi[b]).wait()

    def _issue_in(c, b):
        pltpu.async_copy(hw_hbm.at[srcb[b]], rows[b], semg[b])
        pltpu.async_copy(estat_hbm.at[pl.ds(ebase + c * _CHK, _CHK)],
                         est[b], seme[b])

    def _wait_in(c, b):
        pltpu.make_async_copy(hw_hbm.at[srcb[b]], rows[b], semg[b]).wait()
        pltpu.make_async_copy(estat_hbm.at[pl.ds(ebase + c * _CHK, _CHK)],
                              est[b], seme[b]).wait()

    def _compute_scat(c, b):
        def _row(r, cc):
            for k in range(_DN // 16):
                sl = pl.ds(k * 16, 16)
                rows[b][r, sl] = jnp.maximum(rows[b][r, sl] + est[b][r, sl],
                                             0.0)
            return cc

        lax.fori_loop(0, _CHK, _row, 0)
        pltpu.sync_copy(rows[b], acc_sh.at[dstb[b]], add=True)

    _issue_idx(0, 0)
    _issue_idx(1, 1)
    _wait_idx(0, 0)
    _issue_in(0, 0)

    def _pair(p, carry):
        ca = 2 * p
        cb = 2 * p + 1
        _wait_idx(cb, 1)
        _issue_in(cb, 1)
        _wait_in(ca, 0)
        _compute_scat(ca, 0)
        _issue_idx(ca + 2, 0)
        _wait_idx(ca + 2, 0)
        _issue_in(ca + 2, 0)
        _wait_in(cb, 1)
        _compute_scat(cb, 1)
        _issue_idx(cb + 2, 1)
        return carry

    lax.fori_loop(0, (_NCHUNK - 1) // 2, _pair, 0)
    _wait_idx(_NCHUNK, 1)
    _wait_in(_NCHUNK - 1, 0)
    _compute_scat(_NCHUNK - 1, 0)

    plsc.subcore_barrier()
    pltpu.sync_copy(acc_sh.at[pl.ds(sid * _RPT, _RPT)],
                    out_hbm.at[pl.ds(cid * _NPAD + sid * _RPT, _RPT)])


def kernel(node_Z, edge_type, edge_dist, edge_index, node_table, edge_table,
           W_msg, b_msg, W_upd, b_upd, W_fc1, b_fc1, W_fc2, b_fc2):
    pad = jnp.zeros((_CHK,), jnp.int32)
    src = jnp.concatenate([edge_index[0].astype(jnp.int32), pad])
    dst = jnp.concatenate([edge_index[1].astype(jnp.int32), pad])
    w_h = W_msg[:_DN]
    w_e = W_msg[_DN:_DN + _DE]
    w_r = W_msg[_DN + _DE:]
    dist2 = edge_dist.reshape(_E, 1).astype(jnp.float32)
    etype2 = edge_type.reshape(_E, 1).astype(jnp.int32)
    z2 = node_Z.reshape(_N, 1).astype(jnp.int32)

    estat = _estat_call(dist2, etype2, edge_table, w_e, w_r,
                        b_msg.reshape(1, _DN))
    hw0 = _hw0_call(z2, node_table, w_h)
    bu2 = b_upd.reshape(1, _DN)

    hw = hw0
    h_fin = None
    for _layer in range(_NCONV):
        parts = _sc_edge(hw, estat, src, dst)
        p0 = parts[:_N]
        p1 = parts[_NPAD:_NPAD + _N]
        h_fin, hw = _upd_call(p0, p1, W_upd, bu2, w_h)
    out = _read_call(h_fin, W_fc1, b_fc1.reshape(1, _DN),
                     W_fc2, b_fc2.reshape(1, 1))
    return out.reshape(1)


# speedup vs baseline: 1.0281x; 1.0281x over previous
import functools

import jax
import jax.numpy as jnp
from jax import lax
from jax.experimental import pallas as pl
from jax.experimental.pallas import tpu as pltpu
from jax.experimental.pallas import tpu_sc as plsc

_N = 10000
_E = 320000
_DN = 128
_DE = 128
_NDICT = 20
_EDICT = 400
_NC = 150
_CLOW = 0.0
_CHIGH = 30.0
_NCONV = 3

_EB = 2560
_NB = 1000
_NWORK = 32
_EPW = _E // _NWORK
_CHK = 40
_NCHUNK = _EPW // _CHK
_NPAD = 10112
_RPT = _NPAD // 16


def _estat_body(dist_ref, etype_ref, etab_ref, we_ref, wr_ref, bmsg_ref,
                out_ref, te_ref):
    i = pl.program_id(0)

    @pl.when(i == 0)
    def _():
        te_ref[...] = jnp.dot(etab_ref[...], we_ref[...],
                              preferred_element_type=jnp.float32)

    delta = (_CHIGH - _CLOW) / (_NC - 1)
    centers = _CLOW + delta * lax.broadcasted_iota(
        jnp.int32, (1, _NC), 1).astype(jnp.float32)
    d = dist_ref[...]
    z = (d - centers) * (1.0 / delta)
    rbf = jnp.exp(-(z * z))
    r_part = jnp.dot(rbf, wr_ref[...], preferred_element_type=jnp.float32)
    t = etype_ref[...]
    oh = (t == lax.broadcasted_iota(jnp.int32, (1, _EDICT), 1))
    e_part = jnp.dot(oh.astype(jnp.float32), te_ref[...],
                     preferred_element_type=jnp.float32)
    out_ref[...] = r_part + e_part + bmsg_ref[...]


def _estat_call(dist2, etype2, edge_table, w_e, w_r, bmsg2):
    return pl.pallas_call(
        _estat_body,
        grid=(_E // _EB,),
        in_specs=[
            pl.BlockSpec((_EB, 1), lambda i: (i, 0)),
            pl.BlockSpec((_EB, 1), lambda i: (i, 0)),
            pl.BlockSpec((_EDICT, _DN), lambda i: (0, 0)),
            pl.BlockSpec((_DE, _DN), lambda i: (0, 0)),
            pl.BlockSpec((_NC, _DN), lambda i: (0, 0)),
            pl.BlockSpec((1, _DN), lambda i: (0, 0)),
        ],
        out_specs=pl.BlockSpec((_EB, _DN), lambda i: (i, 0)),
        out_shape=jax.ShapeDtypeStruct((_E, _DN), jnp.float32),
        scratch_shapes=[pltpu.VMEM((_EDICT, _DN), jnp.float32)],
    )(dist2, etype2, edge_table, w_e, w_r, bmsg2)


def _hw0_body(z_ref, ntab_ref, wh_ref, out_ref):
    twh = jnp.dot(ntab_ref[...], wh_ref[...],
                  preferred_element_type=jnp.float32)
    oh = (z_ref[...] == lax.broadcasted_iota(jnp.int32, (1, _NDICT), 1))
    out_ref[...] = jnp.dot(oh.astype(jnp.float32), twh,
                           preferred_element_type=jnp.float32)


def _hw0_call(z2, node_table, w_h):
    return pl.pallas_call(
        _hw0_body,
        grid=(_N // _NB,),
        in_specs=[
            pl.BlockSpec((_NB, 1), lambda i: (i, 0)),
            pl.BlockSpec((_NDICT, _DN), lambda i: (0, 0)),
            pl.BlockSpec((_DN, _DN), lambda i: (0, 0)),
        ],
        out_specs=pl.BlockSpec((_NB, _DN), lambda i: (i, 0)),
        out_shape=jax.ShapeDtypeStruct((_N, _DN), jnp.float32),
    )(z2, node_table, w_h)


def _upd_body(p0_ref, p1_ref, wu_ref, bu_ref, wh_ref, h_ref, hw_ref):
    agg = p0_ref[...] + p1_ref[...]
    h = jnp.maximum(
        jnp.dot(agg, wu_ref[...], preferred_element_type=jnp.float32)
        + bu_ref[...], 0.0)
    h_ref[...] = h
    hw_ref[...] = jnp.dot(h, wh_ref[...], preferred_element_type=jnp.float32)


def _upd_call(p0, p1, w_upd, bu2, w_h):
    return pl.pallas_call(
        _upd_body,
        grid=(_N // _NB,),
        in_specs=[
            pl.BlockSpec((_NB, _DN), lambda i: (i, 0)),
            pl.BlockSpec((_NB, _DN), lambda i: (i, 0)),
            pl.BlockSpec((_DN, _DN), lambda i: (0, 0)),
            pl.BlockSpec((1, _DN), lambda i: (0, 0)),
            pl.BlockSpec((_DN, _DN), lambda i: (0, 0)),
        ],
        out_specs=[
            pl.BlockSpec((_NB, _DN), lambda i: (i, 0)),
            pl.BlockSpec((_NB, _DN), lambda i: (i, 0)),
        ],
        out_shape=[
            jax.ShapeDtypeStruct((_N, _DN), jnp.float32),
            jax.ShapeDtypeStruct((_N, _DN), jnp.float32),
        ],
    )(p0, p1, w_upd, bu2, w_h)


def _read_body(h_ref, w1_ref, b1_ref, w2_ref, b2_ref, out_ref):
    i = pl.program_id(0)
    t = jnp.maximum(
        jnp.dot(h_ref[...], w1_ref[...], preferred_element_type=jnp.float32)
        + b1_ref[...], 0.0)
    r = jnp.dot(t, w2_ref[...], preferred_element_type=jnp.float32) + b2_ref[...]
    s = jnp.sum(r)

    @pl.when(i == 0)
    def _():
        out_ref[...] = jnp.zeros((1, 1), jnp.float32)

    out_ref[...] = out_ref[...] + jnp.reshape(s, (1, 1))


def _read_call(h, w_fc1, b1_2, w_fc2, b2_2):
    return pl.pallas_call(
        _read_body,
        grid=(_N // _NB,),
        in_specs=[
            pl.BlockSpec((_NB, _DN), lambda i: (i, 0)),
            pl.BlockSpec((_DN, _DN), lambda i: (0, 0)),
            pl.BlockSpec((1, _DN), lambda i: (0, 0)),
            pl.BlockSpec((_DN, 1), lambda i: (0, 0)),
            pl.BlockSpec((1, 1), lambda i: (0, 0)),
        ],
        out_specs=pl.BlockSpec((1, 1), lambda i: (0, 0)),
        out_shape=jax.ShapeDtypeStruct((1, 1), jnp.float32),
    )(h, w_fc1, b1_2, w_fc2, b2_2)


_sc_mesh = plsc.VectorSubcoreMesh(core_axis_name="c", subcore_axis_name="s")


@functools.partial(
    pl.kernel,
    mesh=_sc_mesh,
    out_type=jax.ShapeDtypeStruct((2 * _NPAD, _DN), jnp.float32),
    scratch_types=[
        pltpu.VMEM((_CHK,), jnp.int32),
        pltpu.VMEM((_CHK,), jnp.int32),
        pltpu.VMEM((_CHK,), jnp.int32),
        pltpu.VMEM((_CHK,), jnp.int32),
        pltpu.VMEM((_CHK,), jnp.int32),
        pltpu.VMEM((_CHK,), jnp.int32),
        pltpu.VMEM((_CHK,), jnp.int32),
        pltpu.VMEM((_CHK,), jnp.int32),
        pltpu.VMEM((_CHK, _DN), jnp.float32),
        pltpu.VMEM((_CHK, _DN), jnp.float32),
        pltpu.VMEM((_CHK, _DN), jnp.float32),
        pltpu.VMEM((_CHK, _DN), jnp.float32),
        pltpu.VMEM((_CHK, _DN), jnp.float32),
        pltpu.VMEM((_CHK, _DN), jnp.float32),
        pltpu.VMEM_SHARED((_NPAD, _DN), jnp.float32),
        pltpu.SemaphoreType.DMA,
        pltpu.SemaphoreType.DMA,
        pltpu.SemaphoreType.DMA,
        pltpu.SemaphoreType.DMA,
        pltpu.SemaphoreType.DMA,
        pltpu.SemaphoreType.DMA,
        pltpu.SemaphoreType.DMA,
        pltpu.SemaphoreType.DMA,
        pltpu.SemaphoreType.DMA,
        pltpu.SemaphoreType.DMA,
    ],
)
def _sc_edge(hw_hbm, estat_hbm, src1_hbm, dst1_hbm, out_hbm,
             src0_v, src1_v, src2_v, src3_v, dst0_v, dst1_v, dst2_v, dst3_v,
             rows0, rows1, est0, est1, m0, m1, acc_sh,
             semi0, semi1, semi2, semi3, semg0, semg1, seme0, seme1,
             sems0, sems1):
    cid = lax.axis_index("c")
    sid = lax.axis_index("s")
    wid = cid * 16 + sid
    ebase = wid * _EPW
    rows = (rows0, rows1)
    est = (est0, est1)
    m = (m0, m1)
    srcb = (src0_v, src1_v, src2_v, src3_v)
    dstb = (dst0_v, dst1_v, dst2_v, dst3_v)
    semi = (semi0, semi1, semi2, semi3)
    semg = (semg0, semg1)
    seme = (seme0, seme1)
    sems = (sems0, sems1)

    def _zfill(buf):
        def _zrow(r, carry):
            for k in range(_DN // 16):
                buf[r, pl.ds(k * 16, 16)] = jnp.zeros((16,), jnp.float32)
            return carry

        lax.fori_loop(0, _CHK, _zrow, 0)

    _zfill(m0)
    _zfill(m1)
    for rep in range(_RPT // _CHK):
        pltpu.sync_copy(m0,
                        acc_sh.at[pl.ds(sid * _RPT + rep * _CHK, _CHK)])
    _ZTAIL = _RPT - (_RPT // _CHK) * _CHK
    if _ZTAIL:
        pltpu.sync_copy(
            m0.at[pl.ds(0, _ZTAIL)],
            acc_sh.at[pl.ds(sid * _RPT + (_RPT // _CHK) * _CHK, _ZTAIL)])
    plsc.subcore_barrier()

    def _issue_idx(c, s):
        pltpu.async_copy(src1_hbm.at[pl.ds(ebase + c * _CHK, _CHK)],
                         srcb[s], semi[s])
        pltpu.async_copy(dst1_hbm.at[pl.ds(ebase + c * _CHK, _CHK)],
                         dstb[s], semi[s])

    def _wait_idx(c, s):
        pltpu.make_async_copy(src1_hbm.at[pl.ds(ebase + c * _CHK, _CHK)],
                              srcb[s], semi[s]).wait()
        pltpu.make_async_copy(dst1_hbm.at[pl.ds(ebase + c * _CHK, _CHK)],
                              dstb[s], semi[s]).wait()

    def _issue_in(c, b, s):
        pltpu.async_copy(hw_hbm.at[srcb[s]], rows[b], semg[b])
        pltpu.async_copy(estat_hbm.at[pl.ds(ebase + c * _CHK, _CHK)],
                         est[b], seme[b])

    def _wait_in(c, b, s):
        pltpu.make_async_copy(hw_hbm.at[srcb[s]], rows[b], semg[b]).wait()
        pltpu.make_async_copy(estat_hbm.at[pl.ds(ebase + c * _CHK, _CHK)],
                              est[b], seme[b]).wait()

    def _compute(b):
        def _row(r, cc):
            for k in range(_DN // 16):
                sl = pl.ds(k * 16, 16)
                m[b][r, sl] = jnp.maximum(rows[b][r, sl] + est[b][r, sl],
                                          0.0)
            return cc

        lax.fori_loop(0, _CHK, _row, 0)

    def _issue_scat(b, s):
        pltpu.async_copy(m[b], acc_sh.at[dstb[s]], sems[b], add=True)

    def _wait_scat(b, s):
        pltpu.make_async_copy(m[b], acc_sh.at[dstb[s]], sems[b]).wait()

    def _section(c, b, s_cur, s_next):
        _wait_in(c, b, s_cur)
        _wait_scat(b, s_next)
        _issue_idx(c + 2, s_next)
        _compute(b)
        _issue_scat(b, s_cur)
        _wait_idx(c + 2, s_next)
        _issue_in(c + 2, b, s_next)

    _issue_idx(0, 0)
    _issue_idx(1, 1)
    _wait_idx(0, 0)
    pltpu.async_copy(m0, acc_sh.at[dst0_v], sems0, add=True)
    pltpu.async_copy(m1, acc_sh.at[dst0_v], sems1, add=True)
    _issue_in(0, 0, 0)
    _wait_idx(1, 1)
    _issue_in(1, 1, 1)

    def _quad(q, carry):
        c = 4 * q
        _section(c, 0, 0, 2)
        _section(c + 1, 1, 1, 3)
        _section(c + 2, 0, 2, 0)
        _section(c + 3, 1, 3, 1)
        return carry

    lax.fori_loop(0, _NCHUNK // 4, _quad, 0)
    _wait_in(_NCHUNK - 2, 0, 0)
    _wait_scat(0, 2)
    _compute(0)
    _issue_scat(0, 0)
    _wait_in(_NCHUNK - 1, 1, 1)
    _wait_scat(1, 3)
    _compute(1)
    _issue_scat(1, 1)
    _wait_scat(0, 0)
    _wait_scat(1, 1)

    plsc.subcore_barrier()
    pltpu.sync_copy(acc_sh.at[pl.ds(sid * _RPT, _RPT)],
                    out_hbm.at[pl.ds(cid * _NPAD + sid * _RPT, _RPT)])


def kernel(node_Z, edge_type, edge_dist, edge_index, node_table, edge_table,
           W_msg, b_msg, W_upd, b_upd, W_fc1, b_fc1, W_fc2, b_fc2):
    pad = jnp.zeros((_CHK,), jnp.int32)
    src = jnp.concatenate([edge_index[0].astype(jnp.int32), pad])
    dst = jnp.concatenate([edge_index[1].astype(jnp.int32), pad])
    w_h = W_msg[:_DN]
    w_e = W_msg[_DN:_DN + _DE]
    w_r = W_msg[_DN + _DE:]
    dist2 = edge_dist.reshape(_E, 1).astype(jnp.float32)
    etype2 = edge_type.reshape(_E, 1).astype(jnp.int32)
    z2 = node_Z.reshape(_N, 1).astype(jnp.int32)

    estat = _estat_call(dist2, etype2, edge_table, w_e, w_r,
                        b_msg.reshape(1, _DN))
    hw0 = _hw0_call(z2, node_table, w_h)
    bu2 = b_upd.reshape(1, _DN)

    hw = hw0
    h_fin = None
    for _layer in range(_NCONV):
        parts = _sc_edge(hw, estat, src, dst)
        p0 = parts[:_N]
        p1 = parts[_NPAD:_NPAD + _N]
        h_fin, hw = _upd_call(p0, p1, W_upd, bu2, w_h)
    out = _read_call(h_fin, W_fc1, b_fc1.reshape(1, _DN),
                     W_fc2, b_fc2.reshape(1, 1))
    return out.reshape(1)
